# Initial kernel scaffold; baseline (speedup 1.0000x reference)
import jax
import jax.numpy as jnp
from jax.experimental import pallas as pl


def kernel(adj_indices, adj_values, features, weight):
    raise NotImplementedError("write your pallas kernel here")


# speedup vs baseline: 2.3119x; 2.3119x over previous
import functools

import jax
import jax.numpy as jnp
from jax import lax
from jax.experimental import pallas as pl
from jax.experimental.pallas import tpu as pltpu
from jax.experimental.pallas import tpu_sc as plsc

N = 10000
E = 320000
D = 128

NC = 2
NS = 16
L = 16

DH = D // NC
CHUNK = 128
EPS = -(-E // (NS * CHUNK)) * CHUNK
E_PAD = EPS * NS
NCHUNK = EPS // CHUNK
N_PAD = 10240
RPS = N_PAD // NS
ZROWS = 128


@functools.partial(
    pl.kernel,
    mesh=plsc.VectorSubcoreMesh(core_axis_name="c", subcore_axis_name="s"),
    out_type=jax.ShapeDtypeStruct((NC * N_PAD, DH), jnp.float32),
    compiler_params=pltpu.CompilerParams(use_tc_tiling_on_sc=False),
    scratch_types=[
        pltpu.VMEM((CHUNK,), jnp.int32),
        pltpu.VMEM((CHUNK,), jnp.int32),
        pltpu.VMEM((CHUNK,), jnp.float32),
        pltpu.VMEM((CHUNK, DH), jnp.float32),
        pltpu.VMEM_SHARED((N_PAD, DH), jnp.float32),
        pltpu.VMEM((ZROWS, DH), jnp.float32),
        pltpu.VMEM((RPS, DH), jnp.float32),
        pltpu.VMEM((DH,), jnp.float32),
        pltpu.SemaphoreType.DMA,
    ],
)
def _gcn_sc(featR, rows, cols, vals, wflat, out,
            colb, rowb, valb, msgs, acc, zbuf, obuf, wbuf, sem):
    c = lax.axis_index("c")
    s = lax.axis_index("s")
    coff = c * N
    rbase = s * RPS
    ebase = s * EPS

    zero = jnp.zeros((L,), jnp.float32)
    splat_dnums = lax.GatherDimensionNumbers(
        offset_dims=(), collapsed_slice_dims=(0,), start_index_map=(0,))

    def lane_splat(vv, j):
        return lax.gather(vv, jnp.full((L, 1), j, jnp.int32), splat_dnums,
                          (1,), mode=lax.GatherScatterMode.PROMISE_IN_BOUNDS)

    def zero_row(i, carry):
        for q in range(DH // L):
            zbuf[i, pl.ds(q * L, L)] = zero
        return carry

    lax.fori_loop(0, ZROWS, zero_row, 0)
    for i in range(RPS // ZROWS):
        pltpu.sync_copy(zbuf, acc.at[pl.ds(rbase + i * ZROWS, ZROWS), :])
    plsc.subcore_barrier()

    def chunk_body(k, carry):
        base = ebase + k * CHUNK
        pltpu.sync_copy(cols.at[pl.ds(base, CHUNK)], colb)
        pltpu.sync_copy(rows.at[pl.ds(base, CHUNK)], rowb)
        pltpu.sync_copy(vals.at[pl.ds(base, CHUNK)], valb)
        for g in range(CHUNK // L):
            colb[pl.ds(g * L, L)] = colb[pl.ds(g * L, L)] + coff
        pltpu.async_copy(featR.at[colb], msgs, sem).wait()

        def scale_group(g, carry2):
            vv = valb[pl.ds(g * L, L)]
            for j in range(L):
                sp = lane_splat(vv, j)
                je = g * L + j
                for q in range(DH // L):
                    msgs[je, pl.ds(q * L, L)] = msgs[je, pl.ds(q * L, L)] * sp
            return carry2

        lax.fori_loop(0, CHUNK // L, scale_group, 0)
        pltpu.sync_copy(msgs, acc.at[rowb], add=True)
        return carry

    lax.fori_loop(0, NCHUNK, chunk_body, 0)
    plsc.subcore_barrier()

    pltpu.sync_copy(wflat.at[pl.ds(c * DH, DH)], wbuf)
    pltpu.sync_copy(acc.at[pl.ds(rbase, RPS), :], obuf)

    def finish_row(i, carry):
        for q in range(DH // L):
            wq = wbuf[pl.ds(q * L, L)]
            x = obuf[i, pl.ds(q * L, L)] * wq
            obuf[i, pl.ds(q * L, L)] = jnp.maximum(x, 0.0)
        return carry

    lax.fori_loop(0, RPS, finish_row, 0)
    pltpu.sync_copy(obuf, out.at[pl.ds(c * N_PAD + rbase, RPS), :])


def kernel(adj_indices, adj_values, features, weight):
    idx = adj_indices[0].astype(jnp.int32)
    pad = E_PAD - E
    rows = jnp.concatenate([idx[:, 0], jnp.zeros((pad,), jnp.int32)])
    cols = jnp.concatenate([idx[:, 1], jnp.zeros((pad,), jnp.int32)])
    vals = jnp.concatenate([adj_values[0], jnp.zeros((pad,), jnp.float32)])
    featR = features.reshape(N, NC, DH).transpose(1, 0, 2).reshape(NC * N, DH)
    wflat = weight.reshape(D)
    out2 = _gcn_sc(featR, rows, cols, vals, wflat)
    out2 = out2.reshape(NC, N_PAD, DH)[:, :N, :]
    return out2.transpose(1, 0, 2).reshape(N, D)


# speedup vs baseline: 6.0878x; 2.6332x over previous
import functools

import jax
import jax.numpy as jnp
from jax import lax
from jax.experimental import pallas as pl
from jax.experimental.pallas import tpu as pltpu
from jax.experimental.pallas import tpu_sc as plsc

N = 10000
E = 320000
D = 128

NC = 2
NS = 16
L = 16

DH = D // NC
CHUNK = 128
NCHUNK = 158
EPS = NCHUNK * CHUNK
E_PAD = EPS * NS
N_PAD = 10240
RPS = N_PAD // NS
ZROWS = 128
G = CHUNK // L


@functools.partial(
    pl.kernel,
    mesh=plsc.VectorSubcoreMesh(core_axis_name="c", subcore_axis_name="s"),
    out_type=jax.ShapeDtypeStruct((NC * N_PAD, DH), jnp.float32),
    compiler_params=pltpu.CompilerParams(use_tc_tiling_on_sc=False,
                                         needs_layout_passes=False),
    scratch_types=[
        pltpu.VMEM((3, CHUNK), jnp.int32),
        pltpu.VMEM((3, CHUNK), jnp.int32),
        pltpu.VMEM((CHUNK,), jnp.int32),
        pltpu.VMEM((CHUNK,), jnp.int32),
        pltpu.VMEM((CHUNK,), jnp.int32),
        pltpu.VMEM((CHUNK,), jnp.int32),
        pltpu.VMEM((CHUNK, DH), jnp.float32),
        pltpu.VMEM((CHUNK, DH), jnp.float32),
        pltpu.VMEM_SHARED((N_PAD, DH), jnp.float32),
        pltpu.VMEM((ZROWS, DH), jnp.float32),
        pltpu.VMEM((RPS, DH), jnp.float32),
        pltpu.VMEM((DH,), jnp.float32),
        pltpu.SemaphoreType.DMA,
        pltpu.SemaphoreType.DMA,
        pltpu.SemaphoreType.DMA,
        pltpu.SemaphoreType.DMA,
        pltpu.SemaphoreType.DMA,
        pltpu.SemaphoreType.DMA,
    ],
)
def _gcn_sc(featR, meta, wflat, out,
            metab0, metab1, colb0, colb1, rowb0, rowb1, msgs0, msgs1,
            acc, zbuf, obuf, wbuf, msem0, msem1, gsem0, gsem1, ssem0, ssem1):
    c = lax.axis_index("c")
    s = lax.axis_index("s")
    coff = c * N
    rbase = s * RPS
    cbase = s * NCHUNK

    metab = (metab0, metab1)
    colb = (colb0, colb1)
    rowb = (rowb0, rowb1)
    msgs = (msgs0, msgs1)
    msem = (msem0, msem1)
    gsem = (gsem0, gsem1)
    ssem = (ssem0, ssem1)

    zero = jnp.zeros((L,), jnp.float32)
    splat_dnums = lax.GatherDimensionNumbers(
        offset_dims=(), collapsed_slice_dims=(0,), start_index_map=(0,))

    def lane_splat(vv, j):
        return lax.gather(vv, jnp.full((L, 1), j, jnp.int32), splat_dnums,
                          (1,), mode=lax.GatherScatterMode.PROMISE_IN_BOUNDS)

    @plsc.parallel_loop(0, ZROWS, unroll=4)
    def _(i):
        for q in range(DH // L):
            zbuf[i, pl.ds(q * L, L)] = zero

    for i in range(RPS // ZROWS):
        pltpu.sync_copy(zbuf, acc.at[pl.ds(rbase + i * ZROWS, ZROWS), :])
    plsc.subcore_barrier()

    def start_meta(k, b):
        pltpu.async_copy(meta.at[cbase + k], metab[b], msem[b])

    def wait_meta(b):
        pltpu.make_async_copy(meta.at[cbase], metab[b], msem[b]).wait()

    def index_compute(b):
        for g in range(G):
            seg = pl.ds(g * L, L)
            colb[b][seg] = metab[b][1, seg] + coff
            rowb[b][seg] = metab[b][0, seg]

    def start_gather(b):
        pltpu.async_copy(featR.at[colb[b]], msgs[b], gsem[b])

    def wait_gather(b):
        pltpu.make_async_copy(featR.at[colb[b]], msgs[b], gsem[b]).wait()

    def start_scatter(b):
        pltpu.async_copy(msgs[b], acc.at[rowb[b]], ssem[b], add=True)

    def wait_scatter(b):
        pltpu.make_async_copy(msgs[b], acc.at[rowb[b]], ssem[b]).wait()

    def scale(b):
        @plsc.parallel_loop(0, G, unroll=2)
        def _(g):
            vv = plsc.bitcast(metab[b][2, pl.ds(g * L, L)], jnp.float32)
            for j in range(L):
                sp = lane_splat(vv, j)
                je = g * L + j
                xs = [msgs[b][je, pl.ds(q * L, L)] * sp
                      for q in range(DH // L)]
                for q in range(DH // L):
                    msgs[b][je, pl.ds(q * L, L)] = xs[q]

    start_meta(0, 0)
    start_meta(1, 1)
    wait_meta(0)
    index_compute(0)
    start_gather(0)

    HALF = NCHUNK // 2

    def half_chunk(i, k, b, first):
        def prep():
            wait_meta(1 - b)

            @pl.when(jnp.logical_or(i >= 1, not first))
            def _():
                wait_scatter(1 - b)
            index_compute(1 - b)
            start_gather(1 - b)

        if first:
            prep()
        else:
            @pl.when(i < HALF - 1)
            def _():
                prep()

        wait_gather(b)
        scale(b)

        @pl.when(i < HALF - 1)
        def _():
            start_meta(k + 2, b)

        start_scatter(b)

    def pipe_step(i, carry):
        half_chunk(i, 2 * i, 0, True)
        half_chunk(i, 2 * i + 1, 1, False)
        return carry

    lax.fori_loop(0, HALF, pipe_step, 0)
    wait_scatter(0)
    wait_scatter(1)
    plsc.subcore_barrier()

    pltpu.sync_copy(wflat.at[pl.ds(c * DH, DH)], wbuf)
    pltpu.sync_copy(acc.at[pl.ds(rbase, RPS), :], obuf)

    @plsc.parallel_loop(0, RPS, unroll=4)
    def _(i):
        xs = [obuf[i, pl.ds(q * L, L)] * wbuf[pl.ds(q * L, L)]
              for q in range(DH // L)]
        for q in range(DH // L):
            obuf[i, pl.ds(q * L, L)] = jnp.maximum(xs[q], 0.0)

    pltpu.sync_copy(obuf, out.at[pl.ds(c * N_PAD + rbase, RPS), :])


def kernel(adj_indices, adj_values, features, weight):
    idx = adj_indices[0].astype(jnp.int32)
    pad = E_PAD - E
    rows = jnp.concatenate([idx[:, 0], jnp.zeros((pad,), jnp.int32)])
    cols = jnp.concatenate([idx[:, 1], jnp.zeros((pad,), jnp.int32)])
    vbits = lax.bitcast_convert_type(
        jnp.concatenate([adj_values[0], jnp.zeros((pad,), jnp.float32)]),
        jnp.int32)
    meta = jnp.stack([rows.reshape(-1, CHUNK), cols.reshape(-1, CHUNK),
                      vbits.reshape(-1, CHUNK)], axis=1)
    featR = features.reshape(N, NC, DH).transpose(1, 0, 2).reshape(NC * N, DH)
    wflat = weight.reshape(D)
    out2 = _gcn_sc(featR, meta, wflat)
    out2 = out2.reshape(NC, N_PAD, DH)[:, :N, :]
    return out2.transpose(1, 0, 2).reshape(N, D)


# speedup vs baseline: 7.5100x; 1.2336x over previous
import functools

import jax
import jax.numpy as jnp
from jax import lax
from jax.experimental import pallas as pl
from jax.experimental.pallas import tpu as pltpu
from jax.experimental.pallas import tpu_sc as plsc

N = 10000
E = 320000
D = 128

NC = 2
NS = 16
L = 16

DH = D // NC
CHUNK = 80
NCHUNK = E // (NS * CHUNK)
EPS = NCHUNK * CHUNK
N_PAD = 10240
RPS = N_PAD // NS
ZROWS = 128
G = CHUNK // L
HALF = NCHUNK // 2


@functools.partial(
    pl.kernel,
    mesh=plsc.VectorSubcoreMesh(core_axis_name="c", subcore_axis_name="s"),
    out_type=jax.ShapeDtypeStruct((N, D), jnp.float32),
    compiler_params=pltpu.CompilerParams(use_tc_tiling_on_sc=False,
                                         needs_layout_passes=False),
    scratch_types=[
        pltpu.VMEM((2, CHUNK), jnp.int32),
        pltpu.VMEM((2, CHUNK), jnp.int32),
        pltpu.VMEM((CHUNK,), jnp.float32),
        pltpu.VMEM((CHUNK,), jnp.float32),
        pltpu.VMEM((CHUNK,), jnp.int32),
        pltpu.VMEM((CHUNK,), jnp.int32),
        pltpu.VMEM((CHUNK,), jnp.int32),
        pltpu.VMEM((CHUNK,), jnp.int32),
        pltpu.VMEM((CHUNK, DH), jnp.bfloat16),
        pltpu.VMEM((CHUNK, DH), jnp.bfloat16),
        pltpu.VMEM((CHUNK, DH), jnp.float32),
        pltpu.VMEM((CHUNK, DH), jnp.float32),
        pltpu.VMEM_SHARED((N_PAD, DH), jnp.float32),
        pltpu.VMEM((ZROWS, DH), jnp.float32),
        pltpu.VMEM((RPS, DH), jnp.float32),
        pltpu.VMEM((DH,), jnp.float32),
        pltpu.SemaphoreType.DMA,
        pltpu.SemaphoreType.DMA,
        pltpu.SemaphoreType.DMA,
        pltpu.SemaphoreType.DMA,
        pltpu.SemaphoreType.DMA,
        pltpu.SemaphoreType.DMA,
    ],
)
def _gcn_sc(idxT, vals2, featR, wflat, out,
            islab0, islab1, vslab0, vslab1, colb0, colb1, rowb0, rowb1,
            msgsb0, msgsb1, msgs0, msgs1, acc, zbuf, obuf, wbuf,
            msem0, msem1, gsem0, gsem1, ssem0, ssem1):
    c = lax.axis_index("c")
    s = lax.axis_index("s")
    rbase = s * RPS
    ebase = s * EPS

    islab = (islab0, islab1)
    vslab = (vslab0, vslab1)
    colb = (colb0, colb1)
    rowb = (rowb0, rowb1)
    msgsb = (msgsb0, msgsb1)
    msgs = (msgs0, msgs1)
    msem = (msem0, msem1)
    gsem = (gsem0, gsem1)
    ssem = (ssem0, ssem1)

    zero = jnp.zeros((L,), jnp.float32)
    lane_iota = lax.broadcasted_iota(jnp.int32, (L,), 0)
    zeros_l = jnp.zeros((L,), jnp.int32)
    ones_l = jnp.ones((L,), jnp.int32)
    splat_dnums = lax.GatherDimensionNumbers(
        offset_dims=(), collapsed_slice_dims=(0,), start_index_map=(0,))

    def lane_splat(vv, j):
        return lax.gather(vv, jnp.full((L, 1), j, jnp.int32), splat_dnums,
                          (1,), mode=lax.GatherScatterMode.PROMISE_IN_BOUNDS)

    @plsc.parallel_loop(0, ZROWS, unroll=4)
    def _(i):
        for q in range(DH // L):
            zbuf[i, pl.ds(q * L, L)] = zero

    for i in range(RPS // ZROWS):
        pltpu.sync_copy(zbuf, acc.at[pl.ds(rbase + i * ZROWS, ZROWS), :])
    plsc.subcore_barrier()

    def start_meta(k, b):
        base = ebase + k * CHUNK
        pltpu.async_copy(idxT.at[:, pl.ds(base, CHUNK)], islab[b], msem[b])
        pltpu.async_copy(vals2.at[0, pl.ds(base, CHUNK)], vslab[b], msem[b])

    def wait_meta(b):
        pltpu.make_async_copy(idxT.at[:, pl.ds(0, CHUNK)], islab[b],
                              msem[b]).wait()
        pltpu.make_async_copy(vals2.at[0, pl.ds(0, CHUNK)], vslab[b],
                              msem[b]).wait()

    def index_compute(b):
        for g in range(G):
            seg = pl.ds(g * L, L)
            colb[b][seg] = islab[b][1, seg] * 2 + c
            rowb[b][seg] = islab[b][0, seg]

    def start_gather(b):
        pltpu.async_copy(featR.at[colb[b]], msgsb[b], gsem[b])

    def wait_gather(b):
        pltpu.make_async_copy(featR.at[colb[b]], msgsb[b], gsem[b]).wait()

    def start_scatter(b):
        pltpu.async_copy(msgs[b], acc.at[rowb[b]], ssem[b], add=True)

    def wait_scatter(b):
        pltpu.make_async_copy(msgs[b], acc.at[rowb[b]], ssem[b]).wait()

    def scale(b):
        @plsc.parallel_loop(0, G, unroll=2)
        def _(g):
            vv = vslab[b][pl.ds(g * L, L)]
            for j in range(L):
                sp = lane_splat(vv, j)
                je = g * L + j
                xs = []
                for h in range(DH // (2 * L)):
                    packed = msgsb[b][je, pl.ds(h * 2 * L, 2 * L)]
                    a, bb = plsc.unpack(packed,
                                        format=plsc.PackFormat.INTERLEAVED)
                    xs += [a * sp, bb * sp]
                for q in range(DH // L):
                    msgs[b][je, pl.ds(q * L, L)] = xs[q]

    start_meta(0, 0)
    start_meta(1, 1)
    wait_meta(0)
    index_compute(0)
    start_gather(0)

    def half_chunk(i, k, b, first):
        def prep():
            wait_meta(1 - b)

            @pl.when(jnp.logical_or(i >= 1, not first))
            def _():
                wait_scatter(1 - b)
            index_compute(1 - b)
            start_gather(1 - b)

        if first:
            prep()
        else:
            @pl.when(i < HALF - 1)
            def _():
                prep()

        wait_gather(b)
        scale(b)

        @pl.when(i < HALF - 1)
        def _():
            start_meta(k + 2, b)

        start_scatter(b)

    def pipe_step(i, carry):
        half_chunk(i, 2 * i, 0, True)
        half_chunk(i, 2 * i + 1, 1, False)
        return carry

    lax.fori_loop(0, HALF, pipe_step, 0)
    wait_scatter(0)
    wait_scatter(1)
    plsc.subcore_barrier()

    obase = jnp.minimum(rbase, N - RPS)
    pltpu.sync_copy(wflat.at[pl.ds(c * DH, DH)], wbuf)
    pltpu.sync_copy(acc.at[pl.ds(obase, RPS), :], obuf)

    @plsc.parallel_loop(0, RPS, unroll=4)
    def _(i):
        xs = [obuf[i, pl.ds(q * L, L)] * wbuf[pl.ds(q * L, L)]
              for q in range(DH // L)]
        for q in range(DH // L):
            obuf[i, pl.ds(q * L, L)] = jnp.maximum(xs[q], 0.0)

    pltpu.sync_copy(obuf, out.at[pl.ds(obase, RPS), pl.ds(c * DH, DH)])


_PERM = []
for _h in range(DH // 32):
    for _j in range(16):
        _PERM += [_h * 32 + _j, _h * 32 + 16 + _j]


def kernel(adj_indices, adj_values, features, weight):
    idxT = adj_indices.reshape(E, 2).T
    featR = (features.astype(jnp.bfloat16)
             .reshape(NC * N, DH)[:, jnp.array(_PERM, jnp.int32)])
    wflat = weight.reshape(D)
    return _gcn_sc(idxT, adj_values, featR, wflat)


# speedup vs baseline: 8.3870x; 1.1168x over previous
import functools

import jax
import jax.numpy as jnp
from jax import lax
from jax.experimental import pallas as pl
from jax.experimental.pallas import tpu as pltpu
from jax.experimental.pallas import tpu_sc as plsc

N = 10000
E = 320000
D = 128

NC = 2
NS = 16
L = 16

DH = D // NC
CHUNK = 80
NCHUNK = E // (NS * CHUNK)
EPS = NCHUNK * CHUNK
N_PAD = 10240
RPS = N_PAD // NS
ZROWS = 128
G = CHUNK // L
HA = 48
HB = CHUNK - HA
HALF = NCHUNK // 2


@functools.partial(
    pl.kernel,
    mesh=plsc.VectorSubcoreMesh(core_axis_name="c", subcore_axis_name="s"),
    out_type=jax.ShapeDtypeStruct((N, D), jnp.float32),
    compiler_params=pltpu.CompilerParams(use_tc_tiling_on_sc=False,
                                         needs_layout_passes=False),
    scratch_types=[
        pltpu.VMEM((2, CHUNK), jnp.int32),
        pltpu.VMEM((2, CHUNK), jnp.int32),
        pltpu.VMEM((CHUNK,), jnp.float32),
        pltpu.VMEM((CHUNK,), jnp.float32),
        pltpu.VMEM((HA,), jnp.int32),
        pltpu.VMEM((HA,), jnp.int32),
        pltpu.VMEM((HB,), jnp.int32),
        pltpu.VMEM((HB,), jnp.int32),
        pltpu.VMEM((HA,), jnp.int32),
        pltpu.VMEM((HA,), jnp.int32),
        pltpu.VMEM((HB,), jnp.int32),
        pltpu.VMEM((HB,), jnp.int32),
        pltpu.VMEM((CHUNK, DH), jnp.float32),
        pltpu.VMEM((CHUNK, DH), jnp.float32),
        pltpu.VMEM_SHARED((N_PAD, DH), jnp.float32),
        pltpu.VMEM((ZROWS, DH), jnp.float32),
        pltpu.VMEM((RPS, DH), jnp.float32),
        pltpu.VMEM((DH,), jnp.float32),
        pltpu.SemaphoreType.DMA,
        pltpu.SemaphoreType.DMA,
        pltpu.SemaphoreType.DMA,
        pltpu.SemaphoreType.DMA,
        pltpu.SemaphoreType.DMA,
        pltpu.SemaphoreType.DMA,
    ],
)
def _gcn_sc(idxT, vals2, featR, wflat, out,
            islab0, islab1, vslab0, vslab1,
            colba0, colba1, colbb0, colbb1, rowba0, rowba1, rowbb0, rowbb1,
            msgs0, msgs1, acc, zbuf, obuf, wbuf,
            msem0, msem1, gsem0, gsem1, ssem0, ssem1):
    c = lax.axis_index("c")
    s = lax.axis_index("s")
    rbase = s * RPS
    ebase = s * EPS

    islab = (islab0, islab1)
    vslab = (vslab0, vslab1)
    colba = (colba0, colba1)
    colbb = (colbb0, colbb1)
    rowba = (rowba0, rowba1)
    rowbb = (rowbb0, rowbb1)
    msgs = (msgs0, msgs1)
    msem = (msem0, msem1)
    gsem = (gsem0, gsem1)
    ssem = (ssem0, ssem1)

    zero = jnp.zeros((L,), jnp.float32)
    lane_iota = lax.broadcasted_iota(jnp.int32, (L,), 0)
    zeros_l = jnp.zeros((L,), jnp.int32)
    ones_l = jnp.ones((L,), jnp.int32)
    splat_dnums = lax.GatherDimensionNumbers(
        offset_dims=(), collapsed_slice_dims=(0,), start_index_map=(0,))

    def lane_splat(vv, j):
        return lax.gather(vv, jnp.full((L, 1), j, jnp.int32), splat_dnums,
                          (1,), mode=lax.GatherScatterMode.PROMISE_IN_BOUNDS)

    @plsc.parallel_loop(0, ZROWS, unroll=4)
    def _(i):
        for q in range(DH // L):
            zbuf[i, pl.ds(q * L, L)] = zero

    for i in range(RPS // ZROWS):
        pltpu.sync_copy(zbuf, acc.at[pl.ds(rbase + i * ZROWS, ZROWS), :])
    plsc.subcore_barrier()

    def start_meta(k, b):
        base = ebase + k * CHUNK
        pltpu.async_copy(idxT.at[:, pl.ds(base, CHUNK)], islab[b], msem[b])
        pltpu.async_copy(vals2.at[0, pl.ds(base, CHUNK)], vslab[b], msem[b])

    def wait_meta(b):
        pltpu.make_async_copy(idxT.at[:, pl.ds(0, CHUNK)], islab[b],
                              msem[b]).wait()
        pltpu.make_async_copy(vals2.at[0, pl.ds(0, CHUNK)], vslab[b],
                              msem[b]).wait()

    def index_compute(b):
        for g in range(G):
            seg = pl.ds(g * L, L)
            cols16 = islab[b][1, seg] * 2 + c
            rows16 = islab[b][0, seg]
            if g < HA // L:
                sub = pl.ds(g * L, L)
                colba[b][sub] = cols16
                rowba[b][sub] = rows16
            else:
                sub = pl.ds(g * L - HA, L)
                colbb[b][sub] = cols16
                rowbb[b][sub] = rows16

    def start_gather(b):
        pltpu.async_copy(featR.at[colba[b]], msgs[b].at[pl.ds(0, HA), :],
                         gsem[b])
        pltpu.async_copy(featR.at[colbb[b]], msgs[b].at[pl.ds(HA, HB), :],
                         gsem[b])

    def wait_gather(b):
        pltpu.make_async_copy(featR.at[colba[b]],
                              msgs[b].at[pl.ds(0, HA), :], gsem[b]).wait()
        pltpu.make_async_copy(featR.at[colbb[b]],
                              msgs[b].at[pl.ds(HA, HB), :], gsem[b]).wait()

    def start_scatter(b):
        pltpu.async_copy(msgs[b].at[pl.ds(0, HA), :], acc.at[rowba[b]],
                         ssem[b], add=True)
        pltpu.async_copy(msgs[b].at[pl.ds(HA, HB), :], acc.at[rowbb[b]],
                         ssem[b], add=True)

    def wait_scatter(b):
        pltpu.make_async_copy(msgs[b].at[pl.ds(0, HA), :], acc.at[rowba[b]],
                              ssem[b]).wait()
        pltpu.make_async_copy(msgs[b].at[pl.ds(HA, HB), :], acc.at[rowbb[b]],
                              ssem[b]).wait()

    def scale(b):
        @plsc.parallel_loop(0, G, unroll=2)
        def _(g):
            vv = vslab[b][pl.ds(g * L, L)]
            for j in range(L):
                sp = lane_splat(vv, j)
                je = g * L + j
                xs = [msgs[b][je, pl.ds(q * L, L)] * sp for q in range(DH // L)]
                for q in range(DH // L):
                    msgs[b][je, pl.ds(q * L, L)] = xs[q]

    start_meta(0, 0)
    start_meta(1, 1)
    wait_meta(0)
    index_compute(0)
    start_gather(0)

    def half_chunk(i, k, b, first):
        def prep():
            wait_meta(1 - b)

            @pl.when(jnp.logical_or(i >= 1, not first))
            def _():
                wait_scatter(1 - b)
            index_compute(1 - b)
            start_gather(1 - b)

        if first:
            prep()
        else:
            @pl.when(i < HALF - 1)
            def _():
                prep()

        wait_gather(b)
        scale(b)

        @pl.when(i < HALF - 1)
        def _():
            start_meta(k + 2, b)

        start_scatter(b)

    def pipe_step(i, carry):
        half_chunk(i, 2 * i, 0, True)
        half_chunk(i, 2 * i + 1, 1, False)
        return carry

    lax.fori_loop(0, HALF, pipe_step, 0)
    wait_scatter(0)
    wait_scatter(1)
    plsc.subcore_barrier()

    obase = jnp.minimum(rbase, N - RPS)
    pltpu.sync_copy(wflat.at[pl.ds(c * DH, DH)], wbuf)
    pltpu.sync_copy(acc.at[pl.ds(obase, RPS), :], obuf)

    @plsc.parallel_loop(0, RPS, unroll=4)
    def _(i):
        xs = [obuf[i, pl.ds(q * L, L)] * wbuf[pl.ds(q * L, L)]
              for q in range(DH // L)]
        for q in range(DH // L):
            obuf[i, pl.ds(q * L, L)] = jnp.maximum(xs[q], 0.0)

    pltpu.sync_copy(obuf, out.at[pl.ds(obase, RPS), pl.ds(c * DH, DH)])


def kernel(adj_indices, adj_values, features, weight):
    idxT = adj_indices.reshape(E, 2).T
    featR = features.reshape(NC * N, DH)
    wflat = weight.reshape(D)
    return _gcn_sc(idxT, adj_values, featR, wflat)


# speedup vs baseline: 10.4316x; 1.2438x over previous
import functools

import jax
import jax.numpy as jnp
from jax import lax
from jax.experimental import pallas as pl
from jax.experimental.pallas import tpu as pltpu
from jax.experimental.pallas import tpu_sc as plsc

N = 10000
E = 320000
D = 128

NC = 2
NS = 16
L = 16

CHUNK = 80
EPS = E // (NC * NS)
NCHUNK = EPS // CHUNK
N_PAD = 10240
RPS = N_PAD // NS
ZROWS = 64
G = CHUNK // L
HALF = NCHUNK // 2


@functools.partial(
    pl.kernel,
    mesh=plsc.VectorSubcoreMesh(core_axis_name="c", subcore_axis_name="s"),
    out_type=jax.ShapeDtypeStruct((NC, N_PAD, D), jnp.float32),
    compiler_params=pltpu.CompilerParams(use_tc_tiling_on_sc=False,
                                         needs_layout_passes=False),
    scratch_types=[
        pltpu.VMEM((2, CHUNK), jnp.int32),
        pltpu.VMEM((2, CHUNK), jnp.int32),
        pltpu.VMEM((CHUNK,), jnp.float32),
        pltpu.VMEM((CHUNK,), jnp.float32),
        pltpu.VMEM((CHUNK,), jnp.int32),
        pltpu.VMEM((CHUNK,), jnp.int32),
        pltpu.VMEM((CHUNK,), jnp.int32),
        pltpu.VMEM((CHUNK,), jnp.int32),
        pltpu.VMEM((CHUNK, D), jnp.float32),
        pltpu.VMEM((CHUNK, D), jnp.float32),
        pltpu.VMEM_SHARED((N_PAD, D), jnp.float32),
        pltpu.VMEM((ZROWS, D), jnp.float32),
        pltpu.SemaphoreType.DMA,
        pltpu.SemaphoreType.DMA,
        pltpu.SemaphoreType.DMA,
        pltpu.SemaphoreType.DMA,
        pltpu.SemaphoreType.DMA,
        pltpu.SemaphoreType.DMA,
    ],
)
def _gcn_sc(idxT, vals2, feat, out,
            islab0, islab1, vslab0, vslab1, colb0, colb1, rowb0, rowb1,
            msgs0, msgs1, acc, zbuf,
            msem0, msem1, gsem0, gsem1, ssem0, ssem1):
    c = lax.axis_index("c")
    s = lax.axis_index("s")
    rbase = s * RPS
    ebase = (c * NS + s) * EPS

    islab = (islab0, islab1)
    vslab = (vslab0, vslab1)
    colb = (colb0, colb1)
    rowb = (rowb0, rowb1)
    msgs = (msgs0, msgs1)
    msem = (msem0, msem1)
    gsem = (gsem0, gsem1)
    ssem = (ssem0, ssem1)

    zero = jnp.zeros((L,), jnp.float32)
    splat_dnums = lax.GatherDimensionNumbers(
        offset_dims=(), collapsed_slice_dims=(0,), start_index_map=(0,))

    def lane_splat(vv, j):
        return lax.gather(vv, jnp.full((L, 1), j, jnp.int32), splat_dnums,
                          (1,), mode=lax.GatherScatterMode.PROMISE_IN_BOUNDS)

    @plsc.parallel_loop(0, ZROWS, unroll=4)
    def _(i):
        for q in range(D // L):
            zbuf[i, pl.ds(q * L, L)] = zero

    for i in range(RPS // ZROWS):
        pltpu.sync_copy(zbuf, acc.at[pl.ds(rbase + i * ZROWS, ZROWS), :])
    plsc.subcore_barrier()

    def start_meta(k, b):
        base = ebase + k * CHUNK
        pltpu.async_copy(idxT.at[:, pl.ds(base, CHUNK)], islab[b], msem[b])
        pltpu.async_copy(vals2.at[0, pl.ds(base, CHUNK)], vslab[b], msem[b])

    def wait_meta(b):
        pltpu.make_async_copy(idxT.at[:, pl.ds(0, CHUNK)], islab[b],
                              msem[b]).wait()
        pltpu.make_async_copy(vals2.at[0, pl.ds(0, CHUNK)], vslab[b],
                              msem[b]).wait()

    def index_compute(b):
        for g in range(G):
            seg = pl.ds(g * L, L)
            colb[b][seg] = islab[b][1, seg]
            rowb[b][seg] = islab[b][0, seg]

    def start_gather(b):
        pltpu.async_copy(feat.at[colb[b]], msgs[b], gsem[b])

    def wait_gather(b):
        pltpu.make_async_copy(feat.at[colb[b]], msgs[b], gsem[b]).wait()

    def start_scatter(b):
        pltpu.async_copy(msgs[b], acc.at[rowb[b]], ssem[b], add=True)

    def wait_scatter(b):
        pltpu.make_async_copy(msgs[b], acc.at[rowb[b]], ssem[b]).wait()

    def scale(b):
        @plsc.parallel_loop(0, G, unroll=2)
        def _(g):
            vv = vslab[b][pl.ds(g * L, L)]
            for j in range(L):
                sp = lane_splat(vv, j)
                je = g * L + j
                xs = [msgs[b][je, pl.ds(q * L, L)] * sp for q in range(D // L)]
                for q in range(D // L):
                    msgs[b][je, pl.ds(q * L, L)] = xs[q]

    start_meta(0, 0)
    start_meta(1, 1)
    wait_meta(0)
    index_compute(0)
    start_gather(0)

    def half_chunk(i, k, b, first):
        wait_meta(1 - b)

        @pl.when(jnp.logical_or(i >= 1, not first))
        def _():
            wait_scatter(1 - b)
        index_compute(1 - b)
        start_gather(1 - b)

        wait_gather(b)
        scale(b)

        if first:
            start_meta(k + 2, b)
        else:
            @pl.when(i < HALF - 1)
            def _():
                start_meta(k + 2, b)

        start_scatter(b)

    def pipe_step(i, carry):
        half_chunk(i, 2 * i, 0, True)
        half_chunk(i, 2 * i + 1, 1, False)
        return carry

    lax.fori_loop(0, HALF, pipe_step, 0)
    wait_gather(0)
    scale(0)
    start_scatter(0)
    wait_scatter(1)
    wait_scatter(0)
    plsc.subcore_barrier()

    pltpu.sync_copy(acc.at[pl.ds(rbase, RPS), :],
                    out.at[c, pl.ds(rbase, RPS), :])


def _combine_body(p_ref, w_ref, o_ref):
    o_ref[...] = jax.nn.relu((p_ref[0] + p_ref[1]) * w_ref[...])


_ROWS_BLK = 1024


@jax.jit
def _combine(partials, weight):
    return pl.pallas_call(
        _combine_body,
        grid=(N_PAD // _ROWS_BLK,),
        in_specs=[
            pl.BlockSpec((NC, _ROWS_BLK, D), lambda i: (0, i, 0)),
            pl.BlockSpec((1, D), lambda i: (0, 0)),
        ],
        out_specs=pl.BlockSpec((_ROWS_BLK, D), lambda i: (i, 0)),
        out_shape=jax.ShapeDtypeStruct((N_PAD, D), jnp.float32),
    )(partials, weight)


def kernel(adj_indices, adj_values, features, weight):
    idxT = adj_indices.reshape(E, 2).T
    partials = _gcn_sc(idxT, adj_values, features)
    return _combine(partials, weight)[:N]
